# Initial kernel scaffold; baseline (speedup 1.0000x reference)
import jax
import jax.numpy as jnp
from jax.experimental import pallas as pl


def kernel(_T0, _idx_m, _energy, W, b):
    raise NotImplementedError("write your pallas kernel here")


# speedup vs baseline: 1.4513x; 1.4513x over previous
import functools

import jax
import jax.numpy as jnp
from jax import lax
from jax.experimental import pallas as pl
from jax.experimental.pallas import tpu as pltpu
from jax.experimental.pallas import tpu_sc as plsc

N = 320000
D = 128
M = 10000


FOLD = 16
KW = D * FOLD
NR = N // FOLD
BR = 400
NG = NR // BR


def _mv_body(t_ref, g_ref, b_ref, o_ref):
    y = lax.dot_general(t_ref[...], g_ref[...], (((1,), (0,)), ((), ())),
                        preferred_element_type=jnp.float32)
    o_ref[...] = y + b_ref[0]


def _matvec(t0r, g, b):
    return pl.pallas_call(
        _mv_body,
        grid=(NG,),
        in_specs=[
            pl.BlockSpec((BR, KW), lambda i: (i, 0)),
            pl.BlockSpec((KW, FOLD), lambda i: (0, 0)),
            pl.BlockSpec(memory_space=pltpu.SMEM),
        ],
        out_specs=pl.BlockSpec((BR, FOLD), lambda i: (i, 0)),
        out_shape=jax.ShapeDtypeStruct((NR, FOLD), jnp.float32),
    )(t0r, g, b)


NC = 2
NS = 16
NW = NC * NS
CHUNK = N // NW
NVEC = CHUNK // 16
MP = 10240
RED = MP // NS


def _gather16(x, idx):
    return lax.gather(
        x, idx[:, None],
        dimension_numbers=lax.GatherDimensionNumbers(
            offset_dims=(), collapsed_slice_dims=(0,), start_index_map=(0,)),
        slice_sizes=(1,),
        mode=lax.GatherScatterMode.PROMISE_IN_BOUNDS)


def _sc_body(y_hbm, idx_hbm, idxn_hbm, out_hbm, yv, iv, ivn, tbl, red, acc,
             shared):
    c = lax.axis_index("c")
    s = lax.axis_index("s")
    wid = c * NS + s
    base = wid * CHUNK
    pltpu.sync_copy(y_hbm.at[pl.ds(base, CHUNK)], yv)
    pltpu.sync_copy(idx_hbm.at[pl.ds(base, CHUNK)], iv)
    pltpu.sync_copy(idxn_hbm.at[pl.ds(base, CHUNK)], ivn)

    zero = jnp.zeros((16,), jnp.float32)

    def zbody(i, _):
        tbl[pl.ds(i * 16, 16)] = zero
        return 0

    lax.fori_loop(0, MP // 16, zbody, 0)

    iota = lax.iota(jnp.int32, 16)
    last = iota == 15
    notlast = jnp.logical_not(last)
    shift_idx = [jnp.maximum(iota - sh, 0) for sh in (1, 2, 4, 8)]
    shift_msk = [iota >= sh for sh in (1, 2, 4, 8)]
    fzero = jnp.zeros((16,), jnp.float32)

    def body(k, _):
        off = k * 16
        yk = yv[pl.ds(off, 16)]
        ik = iv[pl.ds(off, 16)]
        ink = ivn[pl.ds(off, 16)]
        sk = yk
        for g_idx, g_msk in zip(shift_idx, shift_msk):
            sk = sk + jnp.where(g_msk, _gather16(sk, g_idx), fzero)
        e = ik != ink
        plsc.addupdate_scatter(tbl, [ik], sk, mask=jnp.logical_or(e, last))
        plsc.addupdate_scatter(tbl, [ink], -sk,
                               mask=jnp.logical_and(e, notlast))
        return 0

    lax.fori_loop(0, NVEC, body, 0)

    pltpu.sync_copy(tbl, shared.at[s])
    plsc.subcore_barrier()

    rbase = s * RED
    for t in range(NS):
        pltpu.sync_copy(shared.at[t, pl.ds(rbase, RED)], red.at[t])

    def rbody(i, _):
        off = i * 16
        v = red[0, pl.ds(off, 16)]

        def tsum(t, vv):
            return vv + red[t, pl.ds(off, 16)]

        acc[pl.ds(off, 16)] = lax.fori_loop(1, NS, tsum, v)
        return 0

    lax.fori_loop(0, RED // 16, rbody, 0)
    pltpu.sync_copy(acc, out_hbm.at[c, pl.ds(rbase, RED)])


_segsum = functools.partial(
    pl.kernel,
    mesh=plsc.VectorSubcoreMesh(core_axis_name="c", subcore_axis_name="s"),
    out_type=jax.ShapeDtypeStruct((NC, MP), jnp.float32),
    compiler_params=pltpu.CompilerParams(needs_layout_passes=False),
    scratch_types=[
        pltpu.VMEM((CHUNK,), jnp.float32),
        pltpu.VMEM((CHUNK,), jnp.int32),
        pltpu.VMEM((CHUNK,), jnp.int32),
        pltpu.VMEM((MP,), jnp.float32),
        pltpu.VMEM((NS, RED), jnp.float32),
        pltpu.VMEM((RED,), jnp.float32),
        pltpu.VMEM_SHARED((NS, MP), jnp.float32),
    ],
)(_sc_body)


MR = MP // 128


def _comb_body(p_ref, e_ref, o_ref):
    o_ref[...] = p_ref[0] + p_ref[1] + e_ref[...]


def _combine(partials, energy_pad):
    return pl.pallas_call(
        _comb_body,
        in_specs=[
            pl.BlockSpec((NC, MR, 128), lambda: (0, 0, 0)),
            pl.BlockSpec((MR, 128), lambda: (0, 0)),
        ],
        out_specs=pl.BlockSpec((MR, 128), lambda: (0, 0)),
        out_shape=jax.ShapeDtypeStruct((MR, 128), jnp.float32),
    )(partials, energy_pad)


def kernel(_T0, _idx_m, _energy, W, b):
    idx32 = _idx_m.astype(jnp.int32)
    idxn = jnp.concatenate([idx32[1:], idx32[-1:]])
    w = W.reshape(D)
    eye = jnp.eye(FOLD, dtype=jnp.float32)
    g = (eye[:, None, :] * w[None, :, None]).reshape(KW, FOLD)
    y = _matvec(_T0.reshape(NR, KW), g, b).reshape(N)
    partials = _segsum(y, idx32, idxn)
    energy_pad = jnp.pad(_energy, (0, MP - M)).reshape(MR, 128)
    out = _combine(partials.reshape(NC, MR, 128), energy_pad)
    return out.reshape(MP)[:M]


# speedup vs baseline: 3.2418x; 2.2338x over previous
import functools

import jax
import jax.numpy as jnp
from jax import lax
from jax.experimental import pallas as pl
from jax.experimental.pallas import tpu as pltpu
from jax.experimental.pallas import tpu_sc as plsc

N = 320000
D = 128
M = 10000


FOLD = 16
KW = D * FOLD
NR = N // FOLD
BR = 400
NG = NR // BR


def _mv_body(t_ref, g_ref, b_ref, o_ref):
    t = t_ref[...].reshape(BR, KW)
    y = lax.dot_general(t, g_ref[...], (((1,), (0,)), ((), ())),
                        preferred_element_type=jnp.float32)
    o_ref[...] = y + b_ref[0]


def _matvec(t0, g, b):
    return pl.pallas_call(
        _mv_body,
        grid=(NG,),
        in_specs=[
            pl.BlockSpec((BR * FOLD, D), lambda i: (i, 0)),
            pl.BlockSpec((KW, FOLD), lambda i: (0, 0)),
            pl.BlockSpec(memory_space=pltpu.SMEM),
        ],
        out_specs=pl.BlockSpec((BR, FOLD), lambda i: (i, 0)),
        out_shape=jax.ShapeDtypeStruct((NR, FOLD), jnp.float32),
    )(t0, g, b)


NC = 2
NS = 16
NW = NC * NS
CHUNK = N // NW
NVEC = CHUNK // 16
MP = 10240
RED = MP // NS


def _gather16(x, idx):
    return lax.gather(
        x, idx[:, None],
        dimension_numbers=lax.GatherDimensionNumbers(
            offset_dims=(), collapsed_slice_dims=(0,), start_index_map=(0,)),
        slice_sizes=(1,),
        mode=lax.GatherScatterMode.PROMISE_IN_BOUNDS)


def _sc_body(y_hbm, idx_hbm, idxn_hbm, out_hbm, yv, iv, ivn, tbl, red, acc,
             shared):
    c = lax.axis_index("c")
    s = lax.axis_index("s")
    wid = c * NS + s
    base = wid * CHUNK
    pltpu.sync_copy(y_hbm.at[pl.ds(base, CHUNK)], yv)
    pltpu.sync_copy(idx_hbm.at[pl.ds(base, CHUNK)], iv)
    pltpu.sync_copy(idxn_hbm.at[pl.ds(base, CHUNK)], ivn)

    zero = jnp.zeros((16,), jnp.float32)

    def zbody(i, _):
        tbl[pl.ds(i * 16, 16)] = zero
        return 0

    lax.fori_loop(0, MP // 16, zbody, 0)

    iota = lax.iota(jnp.int32, 16)
    last = iota == 15
    notlast = jnp.logical_not(last)
    shift_idx = [jnp.maximum(iota - sh, 0) for sh in (1, 2, 4, 8)]
    shift_msk = [iota >= sh for sh in (1, 2, 4, 8)]
    fzero = jnp.zeros((16,), jnp.float32)

    def body(k, _):
        off = k * 16
        yk = yv[pl.ds(off, 16)]
        ik = iv[pl.ds(off, 16)]
        ink = ivn[pl.ds(off, 16)]
        sk = yk
        for g_idx, g_msk in zip(shift_idx, shift_msk):
            sk = sk + jnp.where(g_msk, _gather16(sk, g_idx), fzero)
        e = ik != ink
        plsc.addupdate_scatter(tbl, [ik], sk, mask=jnp.logical_or(e, last))
        plsc.addupdate_scatter(tbl, [ink], -sk,
                               mask=jnp.logical_and(e, notlast))
        return 0

    lax.fori_loop(0, NVEC, body, 0)

    pltpu.sync_copy(tbl, shared.at[s])
    plsc.subcore_barrier()

    rbase = s * RED
    for t in range(NS):
        pltpu.sync_copy(shared.at[t, pl.ds(rbase, RED)], red.at[t])

    def rbody(i, _):
        off = i * 16
        v = red[0, pl.ds(off, 16)]

        def tsum(t, vv):
            return vv + red[t, pl.ds(off, 16)]

        acc[pl.ds(off, 16)] = lax.fori_loop(1, NS, tsum, v)
        return 0

    lax.fori_loop(0, RED // 16, rbody, 0)
    pltpu.sync_copy(acc, out_hbm.at[c, pl.ds(rbase, RED)])


_segsum = functools.partial(
    pl.kernel,
    mesh=plsc.VectorSubcoreMesh(core_axis_name="c", subcore_axis_name="s"),
    out_type=jax.ShapeDtypeStruct((NC, MP), jnp.float32),
    compiler_params=pltpu.CompilerParams(needs_layout_passes=False),
    scratch_types=[
        pltpu.VMEM((CHUNK,), jnp.float32),
        pltpu.VMEM((CHUNK,), jnp.int32),
        pltpu.VMEM((CHUNK,), jnp.int32),
        pltpu.VMEM((MP,), jnp.float32),
        pltpu.VMEM((NS, RED), jnp.float32),
        pltpu.VMEM((RED,), jnp.float32),
        pltpu.VMEM_SHARED((NS, MP), jnp.float32),
    ],
)(_sc_body)


MR = MP // 128


def _comb_body(p_ref, e_ref, o_ref):
    o_ref[...] = p_ref[0] + p_ref[1] + e_ref[...]


def _combine(partials, energy_pad):
    return pl.pallas_call(
        _comb_body,
        in_specs=[
            pl.BlockSpec((NC, MR, 128), lambda: (0, 0, 0)),
            pl.BlockSpec((MR, 128), lambda: (0, 0)),
        ],
        out_specs=pl.BlockSpec((MR, 128), lambda: (0, 0)),
        out_shape=jax.ShapeDtypeStruct((MR, 128), jnp.float32),
    )(partials, energy_pad)


def kernel(_T0, _idx_m, _energy, W, b):
    idx32 = _idx_m.astype(jnp.int32)
    idxn = jnp.concatenate([idx32[1:], idx32[-1:]])
    w = W.reshape(D)
    eye = jnp.eye(FOLD, dtype=jnp.float32)
    g = (eye[:, None, :] * w[None, :, None]).reshape(KW, FOLD)
    y = _matvec(_T0, g, b).reshape(N)
    partials = _segsum(y, idx32, idxn)
    energy_pad = jnp.pad(_energy, (0, MP - M)).reshape(MR, 128)
    out = _combine(partials.reshape(NC, MR, 128), energy_pad)
    return out.reshape(MP)[:M]


# speedup vs baseline: 3.7397x; 1.1536x over previous
import functools

import jax
import jax.numpy as jnp
from jax import lax
from jax.experimental import pallas as pl
from jax.experimental.pallas import tpu as pltpu
from jax.experimental.pallas import tpu_sc as plsc

N = 320000
D = 128
M = 10000


FOLD = 16
KW = D * FOLD
NR = N // FOLD
BR = 800
NG = NR // BR


def _mv_body(t_ref, g_ref, b_ref, o_ref):
    t = t_ref[...].reshape(BR, KW)
    y = lax.dot_general(t, g_ref[...], (((1,), (0,)), ((), ())),
                        preferred_element_type=jnp.float32)
    o_ref[...] = y + b_ref[0]


def _matvec(t0, g, b):
    return pl.pallas_call(
        _mv_body,
        grid=(NG,),
        in_specs=[
            pl.BlockSpec((BR * FOLD, D), lambda i: (i, 0)),
            pl.BlockSpec((KW, FOLD), lambda i: (0, 0)),
            pl.BlockSpec(memory_space=pltpu.SMEM),
        ],
        out_specs=pl.BlockSpec((BR, FOLD), lambda i: (i, 0)),
        out_shape=jax.ShapeDtypeStruct((NR, FOLD), jnp.float32),
    )(t0, g, b)


NC = 2
NS = 16
NW = NC * NS
CHUNK = N // NW
NVEC = CHUNK // 16
MP = 10240
RED = MP // NS


def _gather16(x, idx):
    return lax.gather(
        x, idx[:, None],
        dimension_numbers=lax.GatherDimensionNumbers(
            offset_dims=(), collapsed_slice_dims=(0,), start_index_map=(0,)),
        slice_sizes=(1,),
        mode=lax.GatherScatterMode.PROMISE_IN_BOUNDS)


def _sc_body(y_hbm, idx_hbm, idxn_hbm, out_hbm, yv, iv, ivn, tbl, red, acc,
             shared):
    c = lax.axis_index("c")
    s = lax.axis_index("s")
    wid = c * NS + s
    base = wid * CHUNK
    pltpu.sync_copy(y_hbm.at[pl.ds(base, CHUNK)], yv)
    pltpu.sync_copy(idx_hbm.at[pl.ds(base, CHUNK)], iv)
    pltpu.sync_copy(idxn_hbm.at[pl.ds(base, CHUNK)], ivn)

    zero = jnp.zeros((16,), jnp.float32)

    def zbody(i, _):
        tbl[pl.ds(i * 16, 16)] = zero
        return 0

    lax.fori_loop(0, MP // 16, zbody, 0)

    iota = lax.iota(jnp.int32, 16)
    last = iota == 15
    notlast = jnp.logical_not(last)
    shift_idx = [jnp.maximum(iota - sh, 0) for sh in (1, 2, 4, 8)]
    shift_msk = [iota >= sh for sh in (1, 2, 4, 8)]
    fzero = jnp.zeros((16,), jnp.float32)

    def body(k, _):
        off = k * 16
        yk = yv[pl.ds(off, 16)]
        ik = iv[pl.ds(off, 16)]
        ink = ivn[pl.ds(off, 16)]
        sk = plsc.cumsum(yk)
        e = ik != ink
        plsc.addupdate_scatter(tbl, [ik], sk, mask=jnp.logical_or(e, last))
        plsc.addupdate_scatter(tbl, [ink], -sk,
                               mask=jnp.logical_and(e, notlast))
        return 0

    lax.fori_loop(0, NVEC, body, 0, unroll=5)

    pltpu.sync_copy(tbl, shared.at[s])
    plsc.subcore_barrier()

    rbase = s * RED
    for t in range(NS):
        pltpu.sync_copy(shared.at[t, pl.ds(rbase, RED)], red.at[t])

    def rbody(i, _):
        off = i * 16
        v = red[0, pl.ds(off, 16)]

        def tsum(t, vv):
            return vv + red[t, pl.ds(off, 16)]

        acc[pl.ds(off, 16)] = lax.fori_loop(1, NS, tsum, v)
        return 0

    lax.fori_loop(0, RED // 16, rbody, 0)
    pltpu.sync_copy(acc, out_hbm.at[c, pl.ds(rbase, RED)])


_segsum = functools.partial(
    pl.kernel,
    mesh=plsc.VectorSubcoreMesh(core_axis_name="c", subcore_axis_name="s"),
    out_type=jax.ShapeDtypeStruct((NC, MP), jnp.float32),
    compiler_params=pltpu.CompilerParams(needs_layout_passes=False),
    scratch_types=[
        pltpu.VMEM((CHUNK,), jnp.float32),
        pltpu.VMEM((CHUNK,), jnp.int32),
        pltpu.VMEM((CHUNK,), jnp.int32),
        pltpu.VMEM((MP,), jnp.float32),
        pltpu.VMEM((NS, RED), jnp.float32),
        pltpu.VMEM((RED,), jnp.float32),
        pltpu.VMEM_SHARED((NS, MP), jnp.float32),
    ],
)(_sc_body)


MR = MP // 128


def _comb_body(p_ref, e_ref, o_ref):
    o_ref[...] = p_ref[0] + p_ref[1] + e_ref[...]


def _combine(partials, energy_pad):
    return pl.pallas_call(
        _comb_body,
        in_specs=[
            pl.BlockSpec((NC, MR, 128), lambda: (0, 0, 0)),
            pl.BlockSpec((MR, 128), lambda: (0, 0)),
        ],
        out_specs=pl.BlockSpec((MR, 128), lambda: (0, 0)),
        out_shape=jax.ShapeDtypeStruct((MR, 128), jnp.float32),
    )(partials, energy_pad)


def kernel(_T0, _idx_m, _energy, W, b):
    idx32 = _idx_m.astype(jnp.int32)
    idxn = jnp.concatenate([idx32[1:], idx32[-1:]])
    w = W.reshape(D)
    eye = jnp.eye(FOLD, dtype=jnp.float32)
    g = (eye[:, None, :] * w[None, :, None]).reshape(KW, FOLD)
    y = _matvec(_T0, g, b).reshape(N)
    partials = _segsum(y, idx32, idxn)
    energy_pad = jnp.pad(_energy, (0, MP - M)).reshape(MR, 128)
    out = _combine(partials.reshape(NC, MR, 128), energy_pad)
    return out.reshape(MP)[:M]
